# baseline (speedup 1.0000x reference)
import functools

import jax
import jax.numpy as jnp
from jax import lax
from jax.experimental import pallas as pl
from jax.experimental.pallas import tpu as pltpu
from jax.experimental.pallas import tpu_sc as plsc

_NUM_CORES = 2
_NUM_SUBCORES = 16
_NUM_WORKERS = _NUM_CORES * _NUM_SUBCORES
_LANES = 16


def _desc_token_sums(desc_idx_flat, token_table, B, L, D):
    rows_per_w = B // _NUM_WORKERS
    chunk = 8
    nbuf = 4
    n_chunks = rows_per_w // chunk
    g_rows = chunk * L

    mesh = plsc.VectorSubcoreMesh(
        core_axis_name="c", subcore_axis_name="s",
        num_cores=_NUM_CORES, num_subcores=_NUM_SUBCORES)

    scratch = [pltpu.VMEM((rows_per_w * L,), jnp.int32)]
    scratch += [pltpu.VMEM((g_rows, D), jnp.float32) for _ in range(nbuf)]
    scratch += [pltpu.VMEM((chunk, D), jnp.float32) for _ in range(nbuf)]
    scratch += [pltpu.SemaphoreType.DMA for _ in range(2 * nbuf)]

    @functools.partial(
        pl.kernel,
        mesh=mesh,
        out_type=jax.ShapeDtypeStruct((B, D), jnp.float32),
        scratch_types=scratch,
    )
    def sc_kernel(idx_hbm, table_hbm, desc_out, idx_all, *bufs):
        g_bufs = bufs[0:nbuf]
        o_bufs = bufs[nbuf:2 * nbuf]
        sg = bufs[2 * nbuf:3 * nbuf]
        so = bufs[3 * nbuf:4 * nbuf]
        wid = lax.axis_index("s") * _NUM_CORES + lax.axis_index("c")
        row0 = wid * rows_per_w

        pltpu.sync_copy(idx_hbm.at[pl.ds(row0 * L, rows_per_w * L)], idx_all)

        def fire_gather(c, par):
            pltpu.make_async_copy(
                table_hbm.at[idx_all.at[pl.ds(c * g_rows, g_rows)]],
                g_bufs[par], sg[par]).start()

        for p in range(nbuf):
            fire_gather(p, p)

        def process(c, par):
            out_rows = pl.ds(row0 + c * chunk, chunk)

            pltpu.make_async_copy(
                table_hbm.at[idx_all.at[pl.ds(c * g_rows, g_rows)]],
                g_bufs[par], sg[par]).wait()

            @pl.when(c >= nbuf)
            def _():
                pltpu.make_async_copy(
                    o_bufs[par], desc_out.at[out_rows], so[par]).wait()

            g = g_bufs[par]
            ob = o_bufs[par]

            def row_body(r, carry):
                base = r * L
                for col in range(D // _LANES):
                    sl = pl.ds(col * _LANES, _LANES)
                    acc = g[base, sl]
                    for t in range(1, L):
                        acc = acc + g[base + t, sl]
                    ob[r, sl] = acc
                return carry

            lax.fori_loop(0, chunk, row_body, 0)
            pltpu.make_async_copy(
                ob, desc_out.at[out_rows], so[par]).start()

            @pl.when(c + nbuf < n_chunks)
            def _():
                fire_gather(c + nbuf, par)

        def super_step(s, carry):
            for p in range(nbuf):
                process(s * nbuf + p, p)
            return carry

        lax.fori_loop(0, n_chunks // nbuf, super_step, 0)

        for p in range(nbuf):
            c = n_chunks - nbuf + p
            rows = pl.ds(row0 + c * chunk, chunk)
            pltpu.make_async_copy(
                o_bufs[p], desc_out.at[rows], so[p]).wait()

    return sc_kernel(desc_idx_flat, token_table)


def _combine_tc(ids_col, prop_vector, desc_sums, type_table, prop_W, proj_W,
                proj_b_row, L):
    B, D = desc_sums.shape
    P = prop_vector.shape[1]
    T = type_table.shape[0]
    blk = 2048

    def body(ids_ref, prop_ref, desc_ref, tab_ref, pw_ref, pj_ref, pb_ref,
             out_ref):
        pj = pj_ref[:]
        w1 = pj[:, 0:D]
        w2 = pj[:, D:2 * D]
        w3s = pj[:, 2 * D:3 * D] * (1.0 / L)
        tab_proj = lax.dot_general(
            tab_ref[:], w1, (((1,), (1,)), ((), ())),
            preferred_element_type=jnp.float32)
        fused_prop = lax.dot_general(
            pw_ref[:], w2, (((0,), (1,)), ((), ())),
            preferred_element_type=jnp.float32)
        onehot_t = (ids_ref[0] == lax.broadcasted_iota(jnp.int32, (T, blk), 0)
                    ).astype(jnp.float32)
        out_ref[:] = (
            lax.dot_general(onehot_t, tab_proj, (((0,), (0,)), ((), ())),
                            preferred_element_type=jnp.float32)
            + jnp.dot(prop_ref[:], fused_prop,
                      preferred_element_type=jnp.float32)
            + lax.dot_general(desc_ref[:], w3s, (((1,), (1,)), ((), ())),
                              preferred_element_type=jnp.float32)
            + pb_ref[:])

    return pl.pallas_call(
        body,
        grid=(B // blk,),
        in_specs=[
            pl.BlockSpec((1, 1, blk), lambda i: (i, 0, 0)),
            pl.BlockSpec((blk, P), lambda i: (i, 0)),
            pl.BlockSpec((blk, D), lambda i: (i, 0)),
            pl.BlockSpec((T, D), lambda i: (0, 0)),
            pl.BlockSpec((D, P), lambda i: (0, 0)),
            pl.BlockSpec((D, 3 * D), lambda i: (0, 0)),
            pl.BlockSpec((1, D), lambda i: (0, 0)),
        ],
        out_specs=pl.BlockSpec((blk, D), lambda i: (i, 0)),
        out_shape=jax.ShapeDtypeStruct((B, D), jnp.float32),
    )(ids_col, prop_vector, desc_sums, type_table, prop_W, proj_W,
      proj_b_row)


def kernel(event_type_ids, prop_vector, desc_tokens, event_type_table,
           prop_W, prop_b, token_table, proj_W, proj_b):
    B, L = desc_tokens.shape
    V, D = token_table.shape

    desc_idx_flat = desc_tokens.astype(jnp.int32).reshape(-1)
    desc_sums = _desc_token_sums(desc_idx_flat, token_table, B, L, D)

    ids_col = event_type_ids.astype(jnp.int32).reshape(B // 2048, 1, 2048)
    return _combine_tc(ids_col, prop_vector, desc_sums, event_type_table,
                       prop_W, proj_W, proj_b.reshape(1, D), L)


# speedup vs baseline: 1.1643x; 1.1643x over previous
import functools

import jax
import jax.numpy as jnp
from jax import lax
from jax.experimental import pallas as pl
from jax.experimental.pallas import tpu as pltpu
from jax.experimental.pallas import tpu_sc as plsc

_NUM_CORES = 2
_NUM_SUBCORES = 16
_NUM_WORKERS = _NUM_CORES * _NUM_SUBCORES
_LANES = 16


def _desc_token_sums(desc_idx_flat, token_table, B, L, D):
    rows_per_w = B // _NUM_WORKERS
    chunk = 16
    n_chunks = rows_per_w // chunk
    g_rows = chunk * L

    mesh = plsc.VectorSubcoreMesh(
        core_axis_name="c", subcore_axis_name="s",
        num_cores=_NUM_CORES, num_subcores=_NUM_SUBCORES)

    @functools.partial(
        pl.kernel,
        mesh=mesh,
        out_type=jax.ShapeDtypeStruct((B, D), jnp.float32),
        scratch_types=[
            pltpu.VMEM((rows_per_w * L,), jnp.int32),
            pltpu.VMEM((g_rows, D), jnp.float32),
            pltpu.VMEM((g_rows, D), jnp.float32),
            pltpu.VMEM((chunk, D), jnp.float32),
            pltpu.VMEM((chunk, D), jnp.float32),
            pltpu.SemaphoreType.DMA,
            pltpu.SemaphoreType.DMA,
            pltpu.SemaphoreType.DMA,
            pltpu.SemaphoreType.DMA,
        ],
    )
    def sc_kernel(idx_hbm, table_hbm, desc_out,
                  idx_all, g0, g1, ob0, ob1, sg0, sg1, so0, so1):
        wid = lax.axis_index("s") * _NUM_CORES + lax.axis_index("c")
        row0 = wid * rows_per_w

        g_bufs = (g0, g1)
        o_bufs = (ob0, ob1)
        sg = (sg0, sg1)
        so = (so0, so1)

        pltpu.sync_copy(idx_hbm.at[pl.ds(row0 * L, rows_per_w * L)], idx_all)

        def fire_gather(c, par):
            pltpu.make_async_copy(
                table_hbm.at[idx_all.at[pl.ds(c * g_rows, g_rows)]],
                g_bufs[par], sg[par]).start()

        fire_gather(0, 0)
        fire_gather(1, 1)

        def process(c, par):
            out_rows = pl.ds(row0 + c * chunk, chunk)

            pltpu.make_async_copy(
                table_hbm.at[idx_all.at[pl.ds(c * g_rows, g_rows)]],
                g_bufs[par], sg[par]).wait()

            @pl.when(c >= 2)
            def _():
                pltpu.make_async_copy(
                    o_bufs[par], desc_out.at[out_rows], so[par]).wait()

            g = g_bufs[par]
            ob = o_bufs[par]

            def row_body(r, carry):
                base = r * L
                for col in range(D // _LANES):
                    sl = pl.ds(col * _LANES, _LANES)
                    acc = g[base, sl]
                    for t in range(1, L):
                        acc = acc + g[base + t, sl]
                    ob[r, sl] = acc
                return carry

            lax.fori_loop(0, chunk, row_body, 0)
            pltpu.make_async_copy(
                ob, desc_out.at[out_rows], so[par]).start()

            @pl.when(c + 2 < n_chunks)
            def _():
                fire_gather(c + 2, par)

        def super_step(s, carry):
            process(s * 2, 0)
            process(s * 2 + 1, 1)
            return carry

        lax.fori_loop(0, n_chunks // 2, super_step, 0)

        for par, c in ((0, n_chunks - 2), (1, n_chunks - 1)):
            rows = pl.ds(row0 + c * chunk, chunk)
            pltpu.make_async_copy(
                o_bufs[par], desc_out.at[rows], so[par]).wait()

    return sc_kernel(desc_idx_flat, token_table)


def _combine_tc(ids_col, prop_vector, desc_sums, type_table, prop_W, proj_W,
                proj_b_row, L):
    B, D = desc_sums.shape
    P = prop_vector.shape[1]
    T = type_table.shape[0]
    blk = 4096

    def body(ids_ref, prop_ref, desc_ref, tab_ref, pw_ref, pj_ref, pb_ref,
             out_ref):
        pj = pj_ref[:]
        w1 = pj[:, 0:D]
        w2 = pj[:, D:2 * D]
        w3s = pj[:, 2 * D:3 * D] * (1.0 / L)
        tab_proj = lax.dot_general(
            tab_ref[:], w1, (((1,), (1,)), ((), ())),
            preferred_element_type=jnp.float32)
        fused_prop = lax.dot_general(
            pw_ref[:], w2, (((0,), (1,)), ((), ())),
            preferred_element_type=jnp.float32)
        onehot_t = (ids_ref[0] == lax.broadcasted_iota(jnp.int32, (T, blk), 0)
                    ).astype(jnp.float32)
        out_ref[:] = (
            lax.dot_general(onehot_t, tab_proj, (((0,), (0,)), ((), ())),
                            preferred_element_type=jnp.float32)
            + jnp.dot(prop_ref[:], fused_prop,
                      preferred_element_type=jnp.float32)
            + lax.dot_general(desc_ref[:], w3s, (((1,), (1,)), ((), ())),
                              preferred_element_type=jnp.float32)
            + pb_ref[:])

    return pl.pallas_call(
        body,
        grid=(B // blk,),
        in_specs=[
            pl.BlockSpec((1, 1, blk), lambda i: (i, 0, 0)),
            pl.BlockSpec((blk, P), lambda i: (i, 0)),
            pl.BlockSpec((blk, D), lambda i: (i, 0)),
            pl.BlockSpec((T, D), lambda i: (0, 0)),
            pl.BlockSpec((D, P), lambda i: (0, 0)),
            pl.BlockSpec((D, 3 * D), lambda i: (0, 0)),
            pl.BlockSpec((1, D), lambda i: (0, 0)),
        ],
        out_specs=pl.BlockSpec((blk, D), lambda i: (i, 0)),
        out_shape=jax.ShapeDtypeStruct((B, D), jnp.float32),
    )(ids_col, prop_vector, desc_sums, type_table, prop_W, proj_W,
      proj_b_row)


def kernel(event_type_ids, prop_vector, desc_tokens, event_type_table,
           prop_W, prop_b, token_table, proj_W, proj_b):
    B, L = desc_tokens.shape
    V, D = token_table.shape

    desc_idx_flat = desc_tokens.astype(jnp.int32).reshape(-1)
    desc_sums = _desc_token_sums(desc_idx_flat, token_table, B, L, D)

    ids_col = event_type_ids.astype(jnp.int32).reshape(B // 4096, 1, 4096)
    return _combine_tc(ids_col, prop_vector, desc_sums, event_type_table,
                       prop_W, proj_W, proj_b.reshape(1, D), L)


# speedup vs baseline: 1.1654x; 1.0009x over previous
import functools

import jax
import jax.numpy as jnp
from jax import lax
from jax.experimental import pallas as pl
from jax.experimental.pallas import tpu as pltpu
from jax.experimental.pallas import tpu_sc as plsc

_NUM_CORES = 2
_NUM_SUBCORES = 16
_NUM_WORKERS = _NUM_CORES * _NUM_SUBCORES
_LANES = 16


def _desc_token_sums(desc_idx_flat, token_table, B, L, D):
    rows_per_w = B // _NUM_WORKERS
    chunk = 16
    n_chunks = rows_per_w // chunk
    g_rows = chunk * L

    mesh = plsc.VectorSubcoreMesh(
        core_axis_name="c", subcore_axis_name="s",
        num_cores=_NUM_CORES, num_subcores=_NUM_SUBCORES)

    @functools.partial(
        pl.kernel,
        mesh=mesh,
        out_type=jax.ShapeDtypeStruct((B, D), jnp.float32),
        scratch_types=[
            pltpu.VMEM((rows_per_w * L,), jnp.int32),
            pltpu.VMEM((g_rows, D), jnp.float32),
            pltpu.VMEM((g_rows, D), jnp.float32),
            pltpu.VMEM((chunk, D), jnp.float32),
            pltpu.VMEM((chunk, D), jnp.float32),
            pltpu.SemaphoreType.DMA,
            pltpu.SemaphoreType.DMA,
            pltpu.SemaphoreType.DMA,
            pltpu.SemaphoreType.DMA,
        ],
    )
    def sc_kernel(idx_hbm, table_hbm, desc_out,
                  idx_all, g0, g1, ob0, ob1, sg0, sg1, so0, so1):
        wid = lax.axis_index("s") * _NUM_CORES + lax.axis_index("c")
        row0 = wid * rows_per_w

        g_bufs = (g0, g1)
        o_bufs = (ob0, ob1)
        sg = (sg0, sg1)
        so = (so0, so1)

        pltpu.sync_copy(idx_hbm.at[pl.ds(row0 * L, rows_per_w * L)], idx_all)

        def fire_gather(c, par):
            pltpu.make_async_copy(
                table_hbm.at[idx_all.at[pl.ds(c * g_rows, g_rows)]],
                g_bufs[par], sg[par]).start()

        fire_gather(0, 0)
        fire_gather(1, 1)

        def process(c, par):
            out_rows = pl.ds(row0 + c * chunk, chunk)

            pltpu.make_async_copy(
                table_hbm.at[idx_all.at[pl.ds(c * g_rows, g_rows)]],
                g_bufs[par], sg[par]).wait()

            @pl.when(c >= 2)
            def _():
                pltpu.make_async_copy(
                    o_bufs[par], desc_out.at[out_rows], so[par]).wait()

            g = g_bufs[par]
            ob = o_bufs[par]

            def row_body(r, carry):
                base = r * L
                for col in range(D // _LANES):
                    sl = pl.ds(col * _LANES, _LANES)
                    acc = g[base, sl]
                    for t in range(1, L):
                        acc = acc + g[base + t, sl]
                    ob[r, sl] = acc
                return carry

            lax.fori_loop(0, chunk, row_body, 0)
            pltpu.make_async_copy(
                ob, desc_out.at[out_rows], so[par]).start()

            @pl.when(c + 2 < n_chunks)
            def _():
                fire_gather(c + 2, par)

        def super_step(s, carry):
            process(s * 2, 0)
            process(s * 2 + 1, 1)
            return carry

        lax.fori_loop(0, n_chunks // 2, super_step, 0)

        for par, c in ((0, n_chunks - 2), (1, n_chunks - 1)):
            rows = pl.ds(row0 + c * chunk, chunk)
            pltpu.make_async_copy(
                o_bufs[par], desc_out.at[rows], so[par]).wait()

    return sc_kernel(desc_idx_flat, token_table)


def _combine_tc(ids_col, prop_vector, desc_sums, type_table, prop_W, proj_W,
                proj_b_row, L):
    B, D = desc_sums.shape
    P = prop_vector.shape[1]
    T = type_table.shape[0]
    blk = 8192

    def body(ids_ref, prop_ref, desc_ref, tab_ref, pw_ref, pj_ref, pb_ref,
             out_ref):
        pj = pj_ref[:]
        w1 = pj[:, 0:D]
        w2 = pj[:, D:2 * D]
        w3s = pj[:, 2 * D:3 * D] * (1.0 / L)
        tab_proj = lax.dot_general(
            tab_ref[:], w1, (((1,), (1,)), ((), ())),
            preferred_element_type=jnp.float32)
        fused_prop = lax.dot_general(
            pw_ref[:], w2, (((0,), (1,)), ((), ())),
            preferred_element_type=jnp.float32)
        onehot_t = (ids_ref[0] == lax.broadcasted_iota(jnp.int32, (T, blk), 0)
                    ).astype(jnp.float32)
        out_ref[:] = (
            lax.dot_general(onehot_t, tab_proj, (((0,), (0,)), ((), ())),
                            preferred_element_type=jnp.float32)
            + jnp.dot(prop_ref[:], fused_prop,
                      preferred_element_type=jnp.float32)
            + lax.dot_general(desc_ref[:], w3s, (((1,), (1,)), ((), ())),
                              preferred_element_type=jnp.float32)
            + pb_ref[:])

    return pl.pallas_call(
        body,
        grid=(B // blk,),
        in_specs=[
            pl.BlockSpec((1, 1, blk), lambda i: (i, 0, 0)),
            pl.BlockSpec((blk, P), lambda i: (i, 0)),
            pl.BlockSpec((blk, D), lambda i: (i, 0)),
            pl.BlockSpec((T, D), lambda i: (0, 0)),
            pl.BlockSpec((D, P), lambda i: (0, 0)),
            pl.BlockSpec((D, 3 * D), lambda i: (0, 0)),
            pl.BlockSpec((1, D), lambda i: (0, 0)),
        ],
        out_specs=pl.BlockSpec((blk, D), lambda i: (i, 0)),
        out_shape=jax.ShapeDtypeStruct((B, D), jnp.float32),
    )(ids_col, prop_vector, desc_sums, type_table, prop_W, proj_W,
      proj_b_row)


def kernel(event_type_ids, prop_vector, desc_tokens, event_type_table,
           prop_W, prop_b, token_table, proj_W, proj_b):
    B, L = desc_tokens.shape
    V, D = token_table.shape

    desc_idx_flat = desc_tokens.astype(jnp.int32).reshape(-1)
    desc_sums = _desc_token_sums(desc_idx_flat, token_table, B, L, D)

    ids_col = event_type_ids.astype(jnp.int32).reshape(B // 8192, 1, 8192)
    return _combine_tc(ids_col, prop_vector, desc_sums, event_type_table,
                       prop_W, proj_W, proj_b.reshape(1, D), L)


# speedup vs baseline: 1.2009x; 1.0305x over previous
import functools

import jax
import jax.numpy as jnp
from jax import lax
from jax.experimental import pallas as pl
from jax.experimental.pallas import tpu as pltpu
from jax.experimental.pallas import tpu_sc as plsc

_NUM_CORES = 2
_NUM_SUBCORES = 16
_NUM_WORKERS = _NUM_CORES * _NUM_SUBCORES
_LANES = 16


def _desc_token_sums(desc_tokens, token_table, B, L, D):
    rows_per_w = B // _NUM_WORKERS
    chunk = 16
    n_chunks = rows_per_w // chunk
    g_rows = chunk * L
    stage_rows = rows_per_w // 2

    mesh = plsc.VectorSubcoreMesh(
        core_axis_name="c", subcore_axis_name="s",
        num_cores=_NUM_CORES, num_subcores=_NUM_SUBCORES)

    @functools.partial(
        pl.kernel,
        mesh=mesh,
        out_type=jax.ShapeDtypeStruct((B, D), jnp.float32),
        scratch_types=[
            pltpu.VMEM((rows_per_w * L,), jnp.int32),
            pltpu.VMEM((stage_rows, L), jnp.int32),
            pltpu.VMEM((g_rows, D), jnp.float32),
            pltpu.VMEM((g_rows, D), jnp.float32),
            pltpu.VMEM((chunk, D), jnp.float32),
            pltpu.VMEM((chunk, D), jnp.float32),
            pltpu.SemaphoreType.DMA,
            pltpu.SemaphoreType.DMA,
            pltpu.SemaphoreType.DMA,
            pltpu.SemaphoreType.DMA,
        ],
    )
    def sc_kernel(idx_hbm, table_hbm, desc_out,
                  idx_all, idx_stage, g0, g1, ob0, ob1, sg0, sg1, so0, so1):
        wid = lax.axis_index("s") * _NUM_CORES + lax.axis_index("c")
        row0 = wid * rows_per_w

        g_bufs = (g0, g1)
        o_bufs = (ob0, ob1)
        sg = (sg0, sg1)
        so = (so0, so1)

        for h in range(2):
            pltpu.sync_copy(
                idx_hbm.at[pl.ds(row0 + h * stage_rows, stage_rows), :],
                idx_stage)

            def compact_row(r, carry, h=h):
                flat = (h * stage_rows + r) * L
                idx_all[pl.ds(flat, _LANES)] = idx_stage[r, pl.ds(0, _LANES)]
                idx_all[pl.ds(flat + L - _LANES, _LANES)] = (
                    idx_stage[r, pl.ds(L - _LANES, _LANES)])
                return carry

            lax.fori_loop(0, stage_rows, compact_row, 0)

        def fire_gather(c, par):
            pltpu.make_async_copy(
                table_hbm.at[idx_all.at[pl.ds(c * g_rows, g_rows)]],
                g_bufs[par], sg[par]).start()

        fire_gather(0, 0)
        fire_gather(1, 1)

        def process(c, par):
            out_rows = pl.ds(row0 + c * chunk, chunk)

            pltpu.make_async_copy(
                table_hbm.at[idx_all.at[pl.ds(c * g_rows, g_rows)]],
                g_bufs[par], sg[par]).wait()

            @pl.when(c >= 2)
            def _():
                pltpu.make_async_copy(
                    o_bufs[par], desc_out.at[out_rows], so[par]).wait()

            g = g_bufs[par]
            ob = o_bufs[par]

            def row_body(r, carry):
                base = r * L
                for col in range(D // _LANES):
                    sl = pl.ds(col * _LANES, _LANES)
                    acc = g[base, sl]
                    for t in range(1, L):
                        acc = acc + g[base + t, sl]
                    ob[r, sl] = acc
                return carry

            lax.fori_loop(0, chunk, row_body, 0)
            pltpu.make_async_copy(
                ob, desc_out.at[out_rows], so[par]).start()

            @pl.when(c + 2 < n_chunks)
            def _():
                fire_gather(c + 2, par)

        def super_step(s, carry):
            process(s * 2, 0)
            process(s * 2 + 1, 1)
            return carry

        lax.fori_loop(0, n_chunks // 2, super_step, 0)

        for par, c in ((0, n_chunks - 2), (1, n_chunks - 1)):
            rows = pl.ds(row0 + c * chunk, chunk)
            pltpu.make_async_copy(
                o_bufs[par], desc_out.at[rows], so[par]).wait()

    return sc_kernel(desc_tokens, token_table)


def _combine_tc(ids_col, prop_vector, desc_sums, type_table, prop_W, proj_W,
                proj_b_row, L):
    B, D = desc_sums.shape
    P = prop_vector.shape[1]
    T = type_table.shape[0]
    blk = 8192

    def body(ids_ref, prop_ref, desc_ref, tab_ref, pw_ref, pj_ref, pb_ref,
             out_ref):
        pj = pj_ref[:]
        w1 = pj[:, 0:D]
        w2 = pj[:, D:2 * D]
        w3s = pj[:, 2 * D:3 * D] * (1.0 / L)
        tab_proj = lax.dot_general(
            tab_ref[:], w1, (((1,), (1,)), ((), ())),
            preferred_element_type=jnp.float32)
        fused_prop = lax.dot_general(
            pw_ref[:], w2, (((0,), (1,)), ((), ())),
            preferred_element_type=jnp.float32)
        onehot_t = (ids_ref[0] == lax.broadcasted_iota(jnp.int32, (T, blk), 0)
                    ).astype(jnp.float32)
        out_ref[:] = (
            lax.dot_general(onehot_t, tab_proj, (((0,), (0,)), ((), ())),
                            preferred_element_type=jnp.float32)
            + jnp.dot(prop_ref[:], fused_prop,
                      preferred_element_type=jnp.float32)
            + lax.dot_general(desc_ref[:], w3s, (((1,), (1,)), ((), ())),
                              preferred_element_type=jnp.float32)
            + pb_ref[:])

    return pl.pallas_call(
        body,
        grid=(B // blk,),
        in_specs=[
            pl.BlockSpec((1, 1, blk), lambda i: (i, 0, 0)),
            pl.BlockSpec((blk, P), lambda i: (i, 0)),
            pl.BlockSpec((blk, D), lambda i: (i, 0)),
            pl.BlockSpec((T, D), lambda i: (0, 0)),
            pl.BlockSpec((D, P), lambda i: (0, 0)),
            pl.BlockSpec((D, 3 * D), lambda i: (0, 0)),
            pl.BlockSpec((1, D), lambda i: (0, 0)),
        ],
        out_specs=pl.BlockSpec((blk, D), lambda i: (i, 0)),
        out_shape=jax.ShapeDtypeStruct((B, D), jnp.float32),
    )(ids_col, prop_vector, desc_sums, type_table, prop_W, proj_W,
      proj_b_row)


def kernel(event_type_ids, prop_vector, desc_tokens, event_type_table,
           prop_W, prop_b, token_table, proj_W, proj_b):
    B, L = desc_tokens.shape
    V, D = token_table.shape

    desc_sums = _desc_token_sums(
        desc_tokens.astype(jnp.int32), token_table, B, L, D)

    ids_col = event_type_ids.astype(jnp.int32).reshape(B // 8192, 1, 8192)
    return _combine_tc(ids_col, prop_vector, desc_sums, event_type_table,
                       prop_W, proj_W, proj_b.reshape(1, D), L)


# speedup vs baseline: 1.2092x; 1.0069x over previous
import functools

import jax
import jax.numpy as jnp
from jax import lax
from jax.experimental import pallas as pl
from jax.experimental.pallas import tpu as pltpu
from jax.experimental.pallas import tpu_sc as plsc

_NUM_CORES = 2
_NUM_SUBCORES = 16
_NUM_WORKERS = _NUM_CORES * _NUM_SUBCORES
_LANES = 16


def _desc_token_sums(desc_tokens, token_table, B, L, D):
    rows_per_w = B // _NUM_WORKERS
    chunk = 16
    n_chunks = rows_per_w // chunk
    g_rows = chunk * L
    stage_rows = rows_per_w // 2

    mesh = plsc.VectorSubcoreMesh(
        core_axis_name="c", subcore_axis_name="s",
        num_cores=_NUM_CORES, num_subcores=_NUM_SUBCORES)

    @functools.partial(
        pl.kernel,
        mesh=mesh,
        out_type=jax.ShapeDtypeStruct((B, D), jnp.float32),
        scratch_types=[
            pltpu.VMEM((rows_per_w * L,), jnp.int32),
            pltpu.VMEM((stage_rows, L), jnp.int32),
            pltpu.VMEM((g_rows, D), jnp.float32),
            pltpu.VMEM((g_rows, D), jnp.float32),
            pltpu.VMEM((chunk, D), jnp.float32),
            pltpu.VMEM((chunk, D), jnp.float32),
            pltpu.SemaphoreType.DMA,
            pltpu.SemaphoreType.DMA,
            pltpu.SemaphoreType.DMA,
            pltpu.SemaphoreType.DMA,
        ],
    )
    def sc_kernel(idx_hbm, table_hbm, desc_out,
                  idx_all, idx_stage, g0, g1, ob0, ob1, sg0, sg1, so0, so1):
        wid = lax.axis_index("s") * _NUM_CORES + lax.axis_index("c")
        row0 = wid * rows_per_w

        g_bufs = (g0, g1)
        o_bufs = (ob0, ob1)
        sg = (sg0, sg1)
        so = (so0, so1)

        def fire_gather(c, par):
            pltpu.make_async_copy(
                table_hbm.at[idx_all.at[pl.ds(c * g_rows, g_rows)]],
                g_bufs[par], sg[par]).start()

        def stage_half(h):
            pltpu.sync_copy(
                idx_hbm.at[pl.ds(row0 + h * stage_rows, stage_rows), :],
                idx_stage)

            def compact_row(r, carry):
                flat = (h * stage_rows + r) * L
                idx_all[pl.ds(flat, _LANES)] = idx_stage[r, pl.ds(0, _LANES)]
                idx_all[pl.ds(flat + L - _LANES, _LANES)] = (
                    idx_stage[r, pl.ds(L - _LANES, _LANES)])
                return carry

            lax.fori_loop(0, stage_rows, compact_row, 0)

        stage_half(0)
        fire_gather(0, 0)
        fire_gather(1, 1)
        stage_half(1)

        def process(c, par):
            out_rows = pl.ds(row0 + c * chunk, chunk)

            pltpu.make_async_copy(
                table_hbm.at[idx_all.at[pl.ds(c * g_rows, g_rows)]],
                g_bufs[par], sg[par]).wait()

            @pl.when(c >= 2)
            def _():
                pltpu.make_async_copy(
                    o_bufs[par], desc_out.at[out_rows], so[par]).wait()

            g = g_bufs[par]
            ob = o_bufs[par]

            def row_body(r, carry):
                base = r * L
                for col in range(D // _LANES):
                    sl = pl.ds(col * _LANES, _LANES)
                    acc = g[base, sl]
                    for t in range(1, L):
                        acc = acc + g[base + t, sl]
                    ob[r, sl] = acc
                return carry

            lax.fori_loop(0, chunk, row_body, 0)
            pltpu.make_async_copy(
                ob, desc_out.at[out_rows], so[par]).start()

            @pl.when(c + 2 < n_chunks)
            def _():
                fire_gather(c + 2, par)

        def super_step(s, carry):
            process(s * 2, 0)
            process(s * 2 + 1, 1)
            return carry

        lax.fori_loop(0, n_chunks // 2, super_step, 0)

        for par, c in ((0, n_chunks - 2), (1, n_chunks - 1)):
            rows = pl.ds(row0 + c * chunk, chunk)
            pltpu.make_async_copy(
                o_bufs[par], desc_out.at[rows], so[par]).wait()

    return sc_kernel(desc_tokens, token_table)


def _combine_tc(ids_col, prop_vector, desc_sums, type_table, prop_W, proj_W,
                proj_b_row, L):
    B, D = desc_sums.shape
    P = prop_vector.shape[1]
    T = type_table.shape[0]
    blk = 8192

    def body(ids_ref, prop_ref, desc_ref, tab_ref, pw_ref, pj_ref, pb_ref,
             out_ref):
        pj = pj_ref[:]
        w1 = pj[:, 0:D]
        w2 = pj[:, D:2 * D]
        w3s = pj[:, 2 * D:3 * D] * (1.0 / L)
        tab_proj = lax.dot_general(
            tab_ref[:], w1, (((1,), (1,)), ((), ())),
            preferred_element_type=jnp.float32)
        fused_prop = lax.dot_general(
            pw_ref[:], w2, (((0,), (1,)), ((), ())),
            preferred_element_type=jnp.float32)
        onehot_t = (ids_ref[0] == lax.broadcasted_iota(jnp.int32, (T, blk), 0)
                    ).astype(jnp.float32)
        out_ref[:] = (
            lax.dot_general(onehot_t, tab_proj, (((0,), (0,)), ((), ())),
                            preferred_element_type=jnp.float32)
            + jnp.dot(prop_ref[:], fused_prop,
                      preferred_element_type=jnp.float32)
            + lax.dot_general(desc_ref[:], w3s, (((1,), (1,)), ((), ())),
                              preferred_element_type=jnp.float32)
            + pb_ref[:])

    return pl.pallas_call(
        body,
        grid=(B // blk,),
        in_specs=[
            pl.BlockSpec((1, 1, blk), lambda i: (i, 0, 0)),
            pl.BlockSpec((blk, P), lambda i: (i, 0)),
            pl.BlockSpec((blk, D), lambda i: (i, 0)),
            pl.BlockSpec((T, D), lambda i: (0, 0)),
            pl.BlockSpec((D, P), lambda i: (0, 0)),
            pl.BlockSpec((D, 3 * D), lambda i: (0, 0)),
            pl.BlockSpec((1, D), lambda i: (0, 0)),
        ],
        out_specs=pl.BlockSpec((blk, D), lambda i: (i, 0)),
        out_shape=jax.ShapeDtypeStruct((B, D), jnp.float32),
    )(ids_col, prop_vector, desc_sums, type_table, prop_W, proj_W,
      proj_b_row)


def kernel(event_type_ids, prop_vector, desc_tokens, event_type_table,
           prop_W, prop_b, token_table, proj_W, proj_b):
    B, L = desc_tokens.shape
    V, D = token_table.shape

    desc_sums = _desc_token_sums(
        desc_tokens.astype(jnp.int32), token_table, B, L, D)

    ids_col = event_type_ids.astype(jnp.int32).reshape(B // 8192, 1, 8192)
    return _combine_tc(ids_col, prop_vector, desc_sums, event_type_table,
                       prop_W, proj_W, proj_b.reshape(1, D), L)
